# baseline (speedup 1.0000x reference)
import functools

import jax
import jax.numpy as jnp
from jax.experimental import pallas as pl

_DEPTH = 4
_K = 1024
_D = 64


def _rvq_block(lat_ref, cb_ref, out_ref, idx_ref, *, bt):
    lat = lat_ref[...]
    r = lat
    qsum = jnp.zeros_like(lat)
    iota = jax.lax.broadcasted_iota(jnp.int32, (bt, _K), 1)
    cb_all = cb_ref[...]
    cb_hi = cb_all.astype(jnp.bfloat16)
    res1 = cb_all - cb_hi.astype(jnp.float32)
    cb_mid = res1.astype(jnp.bfloat16)
    res2 = res1 - cb_mid.astype(jnp.float32)
    cb_lo = res2.astype(jnp.bfloat16)
    for level in range(_DEPTH):
        cb = cb_all[level]
        cb_norms = jnp.sum(cb * cb, axis=1)
        r_norms = jnp.sum(r * r, axis=1, keepdims=True)
        cross2 = jax.lax.dot_general(
            r * -2.0, cb,
            dimension_numbers=(((1,), (1,)), ((), ())),
            preferred_element_type=jnp.float32,
        )
        d = (r_norms + cross2) + cb_norms[None, :]
        idx = jnp.argmin(d, axis=1).astype(jnp.int32)
        onehot = (iota == idx[:, None]).astype(jnp.bfloat16)
        dims = (((1,), (0,)), ((), ()))
        qh = jax.lax.dot_general(onehot, cb_hi[level], dimension_numbers=dims,
                                 preferred_element_type=jnp.float32)
        qm = jax.lax.dot_general(onehot, cb_mid[level], dimension_numbers=dims,
                                 preferred_element_type=jnp.float32)
        ql = jax.lax.dot_general(onehot, cb_lo[level], dimension_numbers=dims,
                                 preferred_element_type=jnp.float32)
        q = (qh + qm) + ql
        qsum = qsum + q
        r = r - q
        idx_ref[0, level, :] = idx
    out_ref[...] = lat + (qsum - lat)


def _rvq(latent, codebooks, bt):
    n, d = latent.shape
    nb = n // bt
    out, idx = pl.pallas_call(
        functools.partial(_rvq_block, bt=bt),
        grid=(nb,),
        in_specs=[
            pl.BlockSpec((bt, d), lambda i: (i, 0)),
            pl.BlockSpec((_DEPTH, _K, _D), lambda i: (0, 0, 0)),
        ],
        out_specs=[
            pl.BlockSpec((bt, d), lambda i: (i, 0)),
            pl.BlockSpec((1, _DEPTH, bt), lambda i: (i, 0, 0)),
        ],
        out_shape=[
            jax.ShapeDtypeStruct((n, d), jnp.float32),
            jax.ShapeDtypeStruct((nb, _DEPTH, bt), jnp.int32),
        ],
    )(latent, codebooks)
    return out, idx


def kernel(latent, codebooks):
    bt = 4096
    out, idx = _rvq(latent, codebooks, bt)
    n = latent.shape[0]
    indices = jnp.transpose(idx, (1, 0, 2)).reshape(_DEPTH, n)
    return out, indices


# speedup vs baseline: 1.2156x; 1.2156x over previous
import functools

import jax
import jax.numpy as jnp
from jax.experimental import pallas as pl

_DEPTH = 4
_K = 1024
_D = 64


def _rvq_block(lat_ref, cb_ref, out_ref, idx_ref, *, bt):
    lat = lat_ref[...]
    r = lat
    qsum = jnp.zeros_like(lat)
    iota = jax.lax.broadcasted_iota(jnp.int32, (bt, _K), 1)
    cb_all = cb_ref[...]
    cb_hi = cb_all.astype(jnp.bfloat16)
    res1 = cb_all - cb_hi.astype(jnp.float32)
    cb_mid = res1.astype(jnp.bfloat16)
    res2 = res1 - cb_mid.astype(jnp.float32)
    cb_lo = res2.astype(jnp.bfloat16)
    for level in range(_DEPTH):
        cb = cb_all[level]
        cb_norms = jnp.sum(cb * cb, axis=1)
        r_norms = jnp.sum(r * r, axis=1, keepdims=True)
        cross2 = jax.lax.dot_general(
            r * -2.0, cb,
            dimension_numbers=(((1,), (1,)), ((), ())),
            preferred_element_type=jnp.float32,
        )
        d = (r_norms + cross2) + cb_norms[None, :]
        idx = jnp.argmin(d, axis=1).astype(jnp.int32)
        onehot = (iota == idx[:, None]).astype(jnp.bfloat16)
        dims = (((1,), (0,)), ((), ()))
        qh = jax.lax.dot_general(onehot, cb_hi[level], dimension_numbers=dims,
                                 preferred_element_type=jnp.float32)
        qm = jax.lax.dot_general(onehot, cb_mid[level], dimension_numbers=dims,
                                 preferred_element_type=jnp.float32)
        ql = jax.lax.dot_general(onehot, cb_lo[level], dimension_numbers=dims,
                                 preferred_element_type=jnp.float32)
        q = (qh + qm) + ql
        qsum = qsum + q
        r = r - q
        idx_ref[0, level, :] = idx
    out_ref[...] = lat + (qsum - lat)


def _rvq(latent, codebooks, bt):
    n, d = latent.shape
    nb = n // bt
    out, idx = pl.pallas_call(
        functools.partial(_rvq_block, bt=bt),
        grid=(nb,),
        in_specs=[
            pl.BlockSpec((bt, d), lambda i: (i, 0)),
            pl.BlockSpec((_DEPTH, _K, _D), lambda i: (0, 0, 0)),
        ],
        out_specs=[
            pl.BlockSpec((bt, d), lambda i: (i, 0)),
            pl.BlockSpec((1, _DEPTH, bt), lambda i: (i, 0, 0)),
        ],
        out_shape=[
            jax.ShapeDtypeStruct((n, d), jnp.float32),
            jax.ShapeDtypeStruct((nb, _DEPTH, bt), jnp.int32),
        ],
    )(latent, codebooks)
    return out, idx


def kernel(latent, codebooks):
    bt = 1024
    out, idx = _rvq(latent, codebooks, bt)
    n = latent.shape[0]
    indices = jnp.transpose(idx, (1, 0, 2)).reshape(_DEPTH, n)
    return out, indices


# speedup vs baseline: 2.2369x; 1.8401x over previous
import functools

import jax
import jax.numpy as jnp
from jax.experimental import pallas as pl

_DEPTH = 4
_K = 1024
_D = 64


def _rvq_block(lat_ref, cb_ref, out_ref, idx_ref, *, bt):
    lat = lat_ref[...]
    r = lat
    qsum = jnp.zeros_like(lat)
    iota = jax.lax.broadcasted_iota(jnp.int32, (bt, _K), 1)
    cb_all = cb_ref[...]
    cb_hi = cb_all.astype(jnp.bfloat16)
    res1 = cb_all - cb_hi.astype(jnp.float32)
    cb_mid = res1.astype(jnp.bfloat16)
    res2 = res1 - cb_mid.astype(jnp.float32)
    cb_lo = res2.astype(jnp.bfloat16)
    cb_cat = jnp.concatenate([cb_hi, cb_mid, cb_lo], axis=-1)
    for level in range(_DEPTH):
        cb = cb_all[level]
        cb_norms = jnp.sum(cb * cb, axis=1)
        r_norms = jnp.sum(r * r, axis=1, keepdims=True)
        cross2 = jax.lax.dot_general(
            r * -2.0, cb,
            dimension_numbers=(((1,), (1,)), ((), ())),
            preferred_element_type=jnp.float32,
        )
        d = (r_norms + cross2) + cb_norms[None, :]
        idx = jnp.argmin(d, axis=1).astype(jnp.int32)
        onehot = (iota == idx[:, None]).astype(jnp.bfloat16)
        dims = (((1,), (0,)), ((), ()))
        q3 = jax.lax.dot_general(onehot, cb_cat[level], dimension_numbers=dims,
                                 preferred_element_type=jnp.float32)
        q = (q3[:, :_D] + q3[:, _D:2 * _D]) + q3[:, 2 * _D:]
        qsum = qsum + q
        r = r - q
        idx_ref[0, level, :] = idx
    out_ref[...] = lat + (qsum - lat)


def _rvq(latent, codebooks, bt):
    n, d = latent.shape
    nb = n // bt
    out, idx = pl.pallas_call(
        functools.partial(_rvq_block, bt=bt),
        grid=(nb,),
        in_specs=[
            pl.BlockSpec((bt, d), lambda i: (i, 0)),
            pl.BlockSpec((_DEPTH, _K, _D), lambda i: (0, 0, 0)),
        ],
        out_specs=[
            pl.BlockSpec((bt, d), lambda i: (i, 0)),
            pl.BlockSpec((1, _DEPTH, bt), lambda i: (i, 0, 0)),
        ],
        out_shape=[
            jax.ShapeDtypeStruct((n, d), jnp.float32),
            jax.ShapeDtypeStruct((nb, _DEPTH, bt), jnp.int32),
        ],
    )(latent, codebooks)
    return out, idx


def kernel(latent, codebooks):
    bt = 2048
    out, idx = _rvq(latent, codebooks, bt)
    n = latent.shape[0]
    indices = jnp.transpose(idx, (1, 0, 2)).reshape(_DEPTH, n)
    return out, indices
